# baseline (speedup 1.0000x reference)
import functools
import jax
import jax.numpy as jnp
from jax import lax
from jax.experimental import pallas as pl
from jax.experimental.pallas import tpu as pltpu
from jax.experimental.pallas import tpu_sc as plsc

_TG = 8
_EG = 4
_H = 8192


def _table_body(emb_ref, w_ref, b_ref, t_ref):
    t_ref[...] = lax.dot_general(
        w_ref[...], emb_ref[...], (((1,), (1,)), ((), ())),
        preferred_element_type=jnp.float32) + b_ref[...]


def _make_table_t(emb, W, b):
    return pl.pallas_call(
        _table_body,
        out_shape=jax.ShapeDtypeStruct((16, 16), jnp.float32),
    )(emb, W, b.reshape(16, 1))


def _sc_gather(table_flat, idsT):
    T, B = idsT.shape
    t_per = T // _TG
    units = t_per * (B // _H)
    mesh = plsc.VectorSubcoreMesh(core_axis_name="c", subcore_axis_name="s")

    @functools.partial(
        pl.kernel, mesh=mesh,
        out_type=jax.ShapeDtypeStruct((T * 16, B), jnp.float32),
        scratch_types=[
            pltpu.VMEM((256,), jnp.float32),
            pltpu.VMEM((_H,), jnp.int32),
            pltpu.VMEM((_H,), jnp.int32),
            pltpu.VMEM((_EG * _H,), jnp.float32),
            pltpu.VMEM((_EG * _H,), jnp.float32),
            pltpu.SemaphoreType.DMA,
            pltpu.SemaphoreType.DMA,
            pltpu.SemaphoreType.DMA,
            pltpu.SemaphoreType.DMA,
        ],
        compiler_params=pltpu.CompilerParams(needs_layout_passes=False),
    )
    def k(table_hbm, ids_hbm, out_hbm, table_v, idsA, idsB, rowsA, rowsB,
          isemA, isemB, osemA, osemB):
        wid = lax.axis_index("s") * 2 + lax.axis_index("c")
        t0 = (wid // _EG) * t_per
        e0 = (wid % _EG) * _EG
        pltpu.sync_copy(table_hbm, table_v)
        ids_bufs = (idsA, idsB)
        rows_bufs = (rowsA, rowsB)
        isems = (isemA, isemB)
        osems = (osemA, osemB)

        def unit_pos(u):
            return t0 + u // 2, (u % 2) * _H

        def ids_start(u, p):
            t, n0 = unit_pos(u)
            pltpu.async_copy(ids_hbm.at[t, pl.ds(n0, _H)], ids_bufs[p],
                             isems[p])

        def ids_wait(p):
            pltpu.make_async_copy(ids_hbm.at[t0, pl.ds(0, _H)], ids_bufs[p],
                                  isems[p]).wait()

        def compute(p):
            ids_v = ids_bufs[p]
            rows_v = rows_bufs[p]

            @pl.loop(0, _H // 64, unroll=1)
            def _(gb):
                idss = [ids_v[pl.ds((gb * 4 + k) * 16, 16)]
                        for k in range(4)]
                for e in range(_EG):
                    e16 = (e0 + e) * 16
                    vs = [plsc.load_gather(table_v, [idss[k] + e16])
                          for k in range(4)]
                    for k in range(4):
                        rows_v[pl.ds(e * _H + (gb * 4 + k) * 16, 16)] = vs[k]

        def out_start(u, p):
            t, n0 = unit_pos(u)
            rows_v = rows_bufs[p]
            for e in range(_EG):
                pltpu.async_copy(
                    rows_v.at[pl.ds(e * _H, _H)],
                    out_hbm.at[t * 16 + e0 + e, pl.ds(n0, _H)],
                    osems[p])

        def out_drain(p):
            for _ in range(_EG):
                pltpu.make_async_copy(
                    rows_bufs[p].at[pl.ds(0, _H)],
                    out_hbm.at[0, pl.ds(0, _H)], osems[p]).wait()

        ids_start(0, 0)
        ids_start(1, 1)
        ids_wait(0)
        compute(0)
        out_start(0, 0)
        ids_wait(1)
        ids_start(2, 0)
        compute(1)
        out_start(1, 1)
        ids_wait(0)

        @pl.loop(1, (units - 2) // 2)
        def _(i):
            u = i * 2
            ids_start(u + 1, 1)
            out_drain(0)
            compute(0)
            out_start(u, 0)
            ids_wait(1)
            ids_start(u + 2, 0)
            out_drain(1)
            compute(1)
            out_start(u + 1, 1)
            ids_wait(0)

        ids_start(units - 1, 1)
        out_drain(0)
        compute(0)
        out_start(units - 2, 0)
        ids_wait(1)
        out_drain(1)
        compute(1)
        out_start(units - 1, 1)
        out_drain(0)
        out_drain(1)

    return k(table_flat, idsT)


def kernel(input_ids, emb, W, b):
    B, T = input_ids.shape
    tableT = _make_table_t(emb, W, b).reshape(256)
    outT = _sc_gather(tableT, input_ids.T)
    return jnp.transpose(outT.reshape(T, 16, B), (2, 0, 1))
